# baseline (speedup 1.0000x reference)
import functools

import jax
import jax.numpy as jnp
from jax import lax
from jax.experimental import pallas as pl
from jax.experimental.pallas import tpu as pltpu
from jax.experimental.pallas import tpu_sc as plsc

N = 4194304
NUM_CORES = 2
NUM_SUBCORES = 16
NUM_WORKERS = NUM_CORES * NUM_SUBCORES
TOKENS_PER_WORKER = N // NUM_WORKERS
CHUNK = 8192
NUM_CHUNKS = TOKENS_PER_WORKER // CHUNK
LANES = 16
UNROLL = 8

_mesh = plsc.VectorSubcoreMesh(core_axis_name="c", subcore_axis_name="s")


@functools.partial(
    pl.kernel,
    mesh=_mesh,
    out_type=jax.ShapeDtypeStruct((4, N), jnp.float32),
    scratch_types=[
        pltpu.VMEM((2, CHUNK), jnp.int32),
        pltpu.VMEM((2, 4, CHUNK), jnp.float32),
        pltpu.SemaphoreType.DMA,
        pltpu.SemaphoreType.DMA,
        pltpu.SemaphoreType.DMA,
        pltpu.SemaphoreType.DMA,
    ],
)
def _seq2tensor_sc(seq_hbm, out_hbm, seq_v, out_v, in_sem0, in_sem1,
                   out_sem0, out_sem1):
    in_sems = (in_sem0, in_sem1)
    out_sems = (out_sem0, out_sem1)
    wid = lax.axis_index("s") * NUM_CORES + lax.axis_index("c")
    base = wid * TOKENS_PER_WORKER

    def start_in(ci, b):
        pltpu.async_copy(
            seq_hbm.at[pl.ds(base + ci * CHUNK, CHUNK)], seq_v.at[b],
            in_sems[b])

    def wait_in(b):
        pltpu.make_async_copy(
            seq_hbm.at[pl.ds(base, CHUNK)], seq_v.at[b], in_sems[b]).wait()

    def start_out(ci, b):
        for c in range(4):
            pltpu.async_copy(
                out_v.at[b, c], out_hbm.at[c, pl.ds(base + ci * CHUNK, CHUNK)],
                out_sems[b])

    def wait_out(b):
        for c in range(4):
            pltpu.make_async_copy(
                out_v.at[b, c], out_hbm.at[c, pl.ds(base, CHUNK)],
                out_sems[b]).wait()

    def compute(b):
        def vec_body(i, carry):
            for u in range(UNROLL):
                idx = pl.ds((i * UNROLL + u) * LANES, LANES)
                s = seq_v[b, idx]
                bg = jnp.where(s == 4, 0.25, 0.0).astype(jnp.float32)
                for c in range(4):
                    out_v[b, c, idx] = jnp.where(s == c, 1.0, bg)
            return carry

        lax.fori_loop(0, CHUNK // (LANES * UNROLL), vec_body, 0)

    start_in(0, 0)
    for ci in range(NUM_CHUNKS):
        b = ci % 2
        wait_in(b)
        if ci + 1 < NUM_CHUNKS:
            start_in(ci + 1, 1 - b)
        if ci >= 2:
            wait_out(b)
        compute(b)
        start_out(ci, b)
    wait_out(0)
    wait_out(1)


def kernel(seq):
    return _seq2tensor_sc(seq)


# speedup vs baseline: 1.2617x; 1.2617x over previous
import functools

import jax
import jax.numpy as jnp
from jax import lax
from jax.experimental import pallas as pl
from jax.experimental.pallas import tpu as pltpu
from jax.experimental.pallas import tpu_sc as plsc

N = 4194304
NUM_CORES = 2
NUM_SUBCORES = 16
NUM_WORKERS = NUM_CORES * NUM_SUBCORES
TOKENS_PER_WORKER = N // NUM_WORKERS
CHUNK = 8192
NUM_CHUNKS = TOKENS_PER_WORKER // CHUNK
LANES = 16
UNROLL = 8

_mesh = plsc.VectorSubcoreMesh(core_axis_name="c", subcore_axis_name="s")


@functools.partial(
    pl.kernel,
    mesh=_mesh,
    out_type=jax.ShapeDtypeStruct((4, N), jnp.float32),
    scratch_types=[
        pltpu.VMEM((2, CHUNK), jnp.int32),
        pltpu.VMEM((2, 4, CHUNK), jnp.float32),
        pltpu.SemaphoreType.DMA,
        pltpu.SemaphoreType.DMA,
        pltpu.SemaphoreType.DMA,
        pltpu.SemaphoreType.DMA,
    ],
)
def _seq2tensor_sc(seq_hbm, out_hbm, seq_v, out_v, in_sem0, in_sem1,
                   out_sem0, out_sem1):
    in_sems = (in_sem0, in_sem1)
    out_sems = (out_sem0, out_sem1)
    wid = lax.axis_index("s") * NUM_CORES + lax.axis_index("c")
    base = wid * TOKENS_PER_WORKER

    def start_in(ci, b):
        pltpu.async_copy(
            seq_hbm.at[pl.ds(base + ci * CHUNK, CHUNK)], seq_v.at[b],
            in_sems[b])

    def wait_in(b):
        pltpu.make_async_copy(
            seq_hbm.at[pl.ds(base, CHUNK)], seq_v.at[b], in_sems[b]).wait()

    def start_out(ci, b):
        for c in range(4):
            pltpu.async_copy(
                out_v.at[b, c], out_hbm.at[c, pl.ds(base + ci * CHUNK, CHUNK)],
                out_sems[b])

    def wait_out(b):
        for c in range(4):
            pltpu.make_async_copy(
                out_v.at[b, c], out_hbm.at[c, pl.ds(base, CHUNK)],
                out_sems[b]).wait()

    def compute(b):
        @plsc.parallel_loop(0, CHUNK, step=LANES, unroll=UNROLL)
        def _(i):
            s = seq_v[b, pl.ds(i, LANES)]
            bg = jnp.where(s == 4, 0.25, 0.0).astype(jnp.float32)
            for c in range(4):
                out_v[b, c, pl.ds(i, LANES)] = jnp.where(s == c, 1.0, bg)

    start_in(0, 0)
    for ci in range(NUM_CHUNKS):
        b = ci % 2
        wait_in(b)
        if ci + 1 < NUM_CHUNKS:
            start_in(ci + 1, 1 - b)
        if ci >= 2:
            wait_out(b)
        compute(b)
        start_out(ci, b)
    wait_out(0)
    wait_out(1)


def kernel(seq):
    return _seq2tensor_sc(seq)
